# Initial kernel scaffold; baseline (speedup 1.0000x reference)
import jax
import jax.numpy as jnp
from jax.experimental import pallas as pl


def kernel(y):
    raise NotImplementedError("write your pallas kernel here")


# speedup vs baseline: 1.6069x; 1.6069x over previous
import numpy as np
import jax
import jax.numpy as jnp
from jax.experimental import pallas as pl

_ALPHA = 10.0
_C = 1000
_B = 16384
_LANES = 1024
_R = 256

_P_C = np.float32(1.0 - (_C - 1) / _C * (_ALPHA / 100.0))
_P_I = np.float32(1.0 / _C * (_ALPHA / 100.0))
_TINY = np.float32(np.finfo(np.float32).tiny)
_SPAN = np.float32(np.float32(1.0) - np.finfo(np.float32).tiny)


def _round4(x0, x1, rots):
    for d in rots:
        x0 = x0 + x1
        x1 = (x1 << jnp.uint32(d)) | (x1 >> jnp.uint32(32 - d))
        x1 = x0 ^ x1
    return x0, x1


def _disturb_block(y_ref, out_ref):
    b = pl.program_id(0)
    y = y_ref[0, 0, :]

    rows = jax.lax.broadcasted_iota(jnp.int32, (_R, _LANES), 0) + b * _R
    cols = jax.lax.broadcasted_iota(jnp.int32, (_R, _LANES), 1)
    j = (rows * _C + cols).astype(jnp.uint32)

    ks0 = jnp.uint32(0)
    ks1 = jnp.uint32(42)
    ks2 = ks0 ^ ks1 ^ jnp.uint32(0x1BD11BDA)
    r_even = (13, 15, 26, 6)
    r_odd = (17, 29, 16, 24)

    x0 = jnp.full_like(j, ks0)
    x1 = j + ks1
    x0, x1 = _round4(x0, x1, r_even)
    x0, x1 = x0 + ks1, x1 + (ks2 + jnp.uint32(1))
    x0, x1 = _round4(x0, x1, r_odd)
    x0, x1 = x0 + ks2, x1 + (ks0 + jnp.uint32(2))
    x0, x1 = _round4(x0, x1, r_even)
    x0, x1 = x0 + ks0, x1 + (ks1 + jnp.uint32(3))
    x0, x1 = _round4(x0, x1, r_odd)
    x0, x1 = x0 + ks1, x1 + (ks2 + jnp.uint32(4))
    x0, x1 = _round4(x0, x1, r_even)
    x0, x1 = x0 + ks2, x1 + (ks0 + jnp.uint32(5))
    bits = x0 ^ x1

    fb = (bits >> jnp.uint32(9)) | jnp.uint32(0x3F800000)
    f = jax.lax.bitcast_convert_type(fb, jnp.float32) - jnp.float32(1.0)
    u = jnp.maximum(_TINY, f * _SPAN + _TINY)
    g = -jnp.log(-jnp.log(u))

    pv = jnp.where(cols == y[:, None], _P_C, _P_I)
    v = jnp.where(cols < _C, g + jnp.log(pv), -jnp.inf)

    m = jnp.max(v, axis=1, keepdims=True)
    cand = jnp.where(v == m, cols, jnp.int32(2**30))
    out_ref[0, 0, :] = jnp.min(cand, axis=1)


def kernel(y):
    nb = _B // _R
    out = pl.pallas_call(
        _disturb_block,
        grid=(nb,),
        in_specs=[pl.BlockSpec((1, 1, _R), lambda i: (i, 0, 0))],
        out_specs=pl.BlockSpec((1, 1, _R), lambda i: (i, 0, 0)),
        out_shape=jax.ShapeDtypeStruct((nb, 1, _R), jnp.int32),
    )(y.reshape(nb, 1, _R))
    return out.reshape(_B)
